# Initial kernel scaffold; baseline (speedup 1.0000x reference)
import jax
import jax.numpy as jnp
from jax.experimental import pallas as pl


def kernel(tokens, embeddings):
    raise NotImplementedError("write your pallas kernel here")


# speedup vs baseline: 2.5471x; 2.5471x over previous
import functools

import jax
import jax.numpy as jnp
from jax import lax
from jax.experimental import pallas as pl
from jax.experimental.pallas import tpu as pltpu
from jax.experimental.pallas import tpu_sc as plsc

N_TOKENS = 3276800
VOCAB = 1000
DIM = 32
LANES = 16

_info = plsc.get_sparse_core_info()
_NC, _NS = _info.num_cores, _info.num_subcores
_NW = _NC * _NS

_B_PER_W = N_TOKENS // _NW
_CHUNK = 1024
_STEPS = _B_PER_W // _CHUNK
_GROUPS = _CHUNK // LANES


def _make_gather():
    mesh = plsc.VectorSubcoreMesh(core_axis_name="c", subcore_axis_name="s")

    @functools.partial(
        pl.kernel,
        mesh=mesh,
        compiler_params=pltpu.CompilerParams(needs_layout_passes=False),
        out_type=jax.ShapeDtypeStruct((N_TOKENS * DIM,), jnp.float32),
        scratch_types=[
            pltpu.VMEM((VOCAB * DIM,), jnp.float32),
            pltpu.VMEM((_CHUNK,), jnp.int32),
            pltpu.VMEM((_CHUNK * DIM,), jnp.float32),
            pltpu.SemaphoreType.DMA,
        ],
    )
    def k(table_hbm, idx_hbm, out_hbm, table_v, idx_v, rows_v, sem):
        wid = lax.axis_index("s") * _NC + lax.axis_index("c")
        base = wid * _B_PER_W

        pltpu.sync_copy(table_hbm, table_v)
        iota32 = lax.iota(jnp.int32, LANES) * DIM

        def step(s, carry):
            off = base + s * _CHUNK
            pltpu.sync_copy(idx_hbm.at[pl.ds(off, _CHUNK)], idx_v)

            def group(g, carry2):
                t = idx_v[pl.ds(g * LANES, LANES)] * DIM
                dst = iota32 + g * (LANES * DIM)
                for c in range(DIM):
                    vals = plsc.load_gather(table_v, [t + c])
                    plsc.store_scatter(rows_v, [dst + c], vals)
                return carry2

            lax.fori_loop(0, _GROUPS, group, 0)
            pltpu.sync_copy(rows_v, out_hbm.at[pl.ds(off * DIM, _CHUNK * DIM)])
            return carry

        lax.fori_loop(0, _STEPS, step, 0)

    return k


_gather = _make_gather()


def kernel(tokens, embeddings):
    return _gather(embeddings.reshape(-1), tokens).reshape(N_TOKENS, DIM)


# speedup vs baseline: 3.2636x; 1.2813x over previous
import functools

import jax
import jax.numpy as jnp
from jax import lax
from jax.experimental import pallas as pl
from jax.experimental.pallas import tpu as pltpu
from jax.experimental.pallas import tpu_sc as plsc

N_TOKENS = 3276800
VOCAB = 1000
DIM = 32
LANES = 16

_info = plsc.get_sparse_core_info()
_NC, _NS = _info.num_cores, _info.num_subcores
_NW = _NC * _NS

_B_PER_W = N_TOKENS // _NW
_CHUNK = 1024
_STEPS = _B_PER_W // _CHUNK
_GROUPS = _CHUNK // LANES


def _make_gather():
    mesh = plsc.VectorSubcoreMesh(core_axis_name="c", subcore_axis_name="s")

    @functools.partial(
        pl.kernel,
        mesh=mesh,
        compiler_params=pltpu.CompilerParams(
            needs_layout_passes=False, use_tc_tiling_on_sc=False
        ),
        out_type=jax.ShapeDtypeStruct((N_TOKENS, DIM), jnp.float32),
        scratch_types=[
            pltpu.VMEM((VOCAB * DIM,), jnp.float32),
            pltpu.VMEM((_CHUNK,), jnp.int32),
            pltpu.VMEM((_CHUNK, DIM), jnp.float32),
            pltpu.SemaphoreType.DMA,
        ],
    )
    def k(table_hbm, idx_hbm, out_hbm, table_v, idx_v, rows_v, sem):
        wid = lax.axis_index("s") * _NC + lax.axis_index("c")
        base = wid * _B_PER_W

        pltpu.sync_copy(table_hbm, table_v)
        iota = lax.iota(jnp.int32, LANES)

        def step(s, carry):
            off = base + s * _CHUNK
            pltpu.sync_copy(idx_hbm.at[pl.ds(off, _CHUNK)], idx_v)

            @plsc.parallel_loop(0, _GROUPS, unroll=4)
            def group(g):
                t32 = idx_v[pl.ds(g * LANES, LANES)] * DIM
                row_ids = iota + g * LANES
                for c in range(DIM):
                    vals = plsc.load_gather(table_v, [t32 + c])
                    c_vec = jnp.full((LANES,), c, jnp.int32)
                    plsc.store_scatter(rows_v, [row_ids, c_vec], vals)

            pltpu.sync_copy(rows_v, out_hbm.at[pl.ds(off, _CHUNK)])
            return carry

        lax.fori_loop(0, _STEPS, step, 0)

    return k


_gather = _make_gather()


def kernel(tokens, embeddings):
    return _gather(embeddings.reshape(-1), tokens)


# speedup vs baseline: 18.5621x; 5.6877x over previous
import functools

import jax
import jax.numpy as jnp
from jax import lax
from jax.experimental import pallas as pl
from jax.experimental.pallas import tpu as pltpu
from jax.experimental.pallas import tpu_sc as plsc

N_TOKENS = 3276800
VOCAB = 1000
DIM = 32
LANES = 16

_info = plsc.get_sparse_core_info()
_NC, _NS = _info.num_cores, _info.num_subcores
_NW = _NC * _NS

_B_PER_W = N_TOKENS // _NW
_CHUNK = 1024
_STEPS = _B_PER_W // _CHUNK
_GROUPS = _CHUNK // LANES
_RG = N_TOKENS // 128
_CG = DIM // 8
_TPC = _CHUNK // 128


def _make_gather():
    mesh = plsc.VectorSubcoreMesh(core_axis_name="c", subcore_axis_name="s")

    @functools.partial(
        pl.kernel,
        mesh=mesh,
        compiler_params=pltpu.CompilerParams(
            needs_layout_passes=False, use_tc_tiling_on_sc=False
        ),
        out_type=jax.ShapeDtypeStruct((_CG, _RG, 8, 128), jnp.float32),
        scratch_types=[
            pltpu.VMEM((VOCAB * DIM,), jnp.float32),
            pltpu.VMEM((_CHUNK,), jnp.int32),
            pltpu.VMEM((_CG, _TPC, 8, 128), jnp.float32),
            pltpu.SemaphoreType.DMA,
        ],
    )
    def k(tab_hbm, idx_hbm, out_hbm, tab_v, idx_v, rows_v, sem):
        wid = lax.axis_index("s") * _NC + lax.axis_index("c")
        base = wid * _B_PER_W

        pltpu.sync_copy(tab_hbm, tab_v)

        def step(s, carry):
            off = base + s * _CHUNK
            pltpu.sync_copy(idx_hbm.at[pl.ds(off, _CHUNK)], idx_v)

            @plsc.parallel_loop(0, _GROUPS, unroll=4)
            def group(g):
                t = idx_v[pl.ds(g * LANES, LANES)]
                j = g // 8
                lane0 = (g % 8) * LANES
                for c in range(DIM):
                    vals = plsc.load_gather(tab_v, [t + c * VOCAB])
                    rows_v[c // 8, j, c % 8, pl.ds(lane0, LANES)] = vals

            b0 = wid * (_B_PER_W // 128) + s * _TPC
            pltpu.sync_copy(rows_v, out_hbm.at[:, pl.ds(b0, _TPC)])
            return carry

        lax.fori_loop(0, _STEPS, step, 0)

    return k


_gather = _make_gather()


def kernel(tokens, embeddings):
    tab_t = embeddings.T.reshape(-1)
    arr4 = _gather(tab_t, tokens)
    out = jnp.transpose(arr4, (1, 3, 0, 2)).reshape(N_TOKENS, DIM)
    return out
